# Initial kernel scaffold; baseline (speedup 1.0000x reference)
import jax
import jax.numpy as jnp
from jax.experimental import pallas as pl


def kernel(feat, sizes_subg, scale, offset):
    raise NotImplementedError("write your pallas kernel here")


# speedup vs baseline: 1.0091x; 1.0091x over previous
import jax
import jax.numpy as jnp
from jax import lax
from jax.experimental import pallas as pl
from jax.experimental.pallas import tpu as pltpu
from jax.experimental.pallas import tpu_sc as plsc

N = 100000
D = 128
L = 16
NJ = D // L
NC, NS = 2, 16
NW = NC * NS
ROWS_PER_W = N // NW
R = 125
G = ROWS_PER_W // R
RD = R * D


def _take16(x, idx):
    dn = lax.GatherDimensionNumbers(
        offset_dims=(), collapsed_slice_dims=(0,), start_index_map=(0,))
    return lax.gather(x, idx[:, None], dn, slice_sizes=(1,),
                      mode=lax.GatherScatterMode.PROMISE_IN_BOUNDS)


def _lanesum(v, perms):
    for p in perms:
        v = v + _take16(v, p)
    return v


def _rsqrt(x):
    i = plsc.bitcast(x, jnp.int32)
    i = jnp.int32(0x5F3759DF) - (i >> 1)
    y = plsc.bitcast(i, jnp.float32)
    xh = x * 0.5
    for _ in range(3):
        y = y * (1.5 - xh * y * y)
    return y


def _body(feat_hbm, scale_hbm, offset_hbm, out_hbm,
          in_v, out_v, so_v, si0, si1, so0, so1):
    c = lax.axis_index("c")
    s = lax.axis_index("s")
    wid = s * NC + c
    base = wid * ROWS_PER_W

    pltpu.sync_copy(scale_hbm, so_v.at[pl.ds(0, D)])
    pltpu.sync_copy(offset_hbm, so_v.at[pl.ds(D, D)])
    sc = [so_v[pl.ds(j * L, L)] for j in range(NJ)]
    of = [so_v[pl.ds(D + j * L, L)] for j in range(NJ)]

    iota = lax.iota(jnp.int32, L)
    perms = [iota ^ 8, iota ^ 4, iota ^ 2, iota ^ 1]
    inv_d = jnp.float32(1.0 / D)
    sin = (si0, si1)
    sout = (so0, so1)

    def in_copy(g, slot):
        return pltpu.make_async_copy(
            feat_hbm.at[pl.ds((base + g * R) * D, RD)],
            in_v.at[pl.ds(slot * RD, RD)], sin[slot])

    def out_copy(g, slot):
        return pltpu.make_async_copy(
            out_v.at[pl.ds(slot * RD, RD)],
            out_hbm.at[pl.ds((base + g * R) * D, RD)], sout[slot])

    def compute(slot):
        ib = slot * RD

        def row(r, carry):
            b = ib + r * D
            v = [in_v[pl.ds(b + j * L, L)] for j in range(NJ)]
            tot = (v[0] + v[1]) + (v[2] + v[3]) + ((v[4] + v[5]) + (v[6] + v[7]))
            mean = _lanesum(tot, perms) * inv_d
            d = [vj - mean for vj in v]
            sq = (d[0] * d[0] + d[1] * d[1]) + (d[2] * d[2] + d[3] * d[3]) + (
                (d[4] * d[4] + d[5] * d[5]) + (d[6] * d[6] + d[7] * d[7]))
            var = _lanesum(sq, perms) * inv_d + 1e-9
            rs = _rsqrt(var)
            for j in range(NJ):
                out_v[pl.ds(b + j * L, L)] = d[j] * (rs * sc[j]) + of[j]
            return carry

        lax.fori_loop(0, R, row, 0)

    in_copy(0, 0).start()

    def pair(i, carry):
        for b in (0, 1):
            g = 2 * i + b
            in_copy(g + 1, 1 - b).start()
            in_copy(g, b).wait()

            @pl.when(g >= 2)
            def _():
                out_copy(g - 2, b).wait()

            compute(b)
            out_copy(g, b).start()
        return carry

    lax.fori_loop(0, (G - 1) // 2, pair, 0)

    g = G - 1
    in_copy(g, 0).wait()
    out_copy(g - 2, 0).wait()
    compute(0)
    out_copy(g, 0).start()
    out_copy(g - 1, 1).wait()
    out_copy(g, 0).wait()


@jax.jit
def _norm(feat_flat, scale_flat, offset_flat):
    mesh = plsc.VectorSubcoreMesh(core_axis_name="c", subcore_axis_name="s")
    f = pl.kernel(
        _body,
        out_type=jax.ShapeDtypeStruct((N * D,), jnp.float32),
        mesh=mesh,
        scratch_types=[
            pltpu.VMEM((2 * RD,), jnp.float32),
            pltpu.VMEM((2 * RD,), jnp.float32),
            pltpu.VMEM((2 * D,), jnp.float32),
            pltpu.SemaphoreType.DMA,
            pltpu.SemaphoreType.DMA,
            pltpu.SemaphoreType.DMA,
            pltpu.SemaphoreType.DMA,
        ],
        compiler_params=pltpu.CompilerParams(needs_layout_passes=False),
    )
    return f(feat_flat, scale_flat, offset_flat)


def kernel(feat, sizes_subg, scale, offset):
    out = _norm(feat.reshape(-1), scale.reshape(-1), offset.reshape(-1))
    return out.reshape(feat.shape)


# speedup vs baseline: 1.0397x; 1.0303x over previous
import jax
import jax.numpy as jnp
from jax import lax
from jax.experimental import pallas as pl
from jax.experimental.pallas import tpu as pltpu
from jax.experimental.pallas import tpu_sc as plsc

N = 100000
D = 128
L = 16
NJ = D // L
NC, NS = 2, 16
NW = NC * NS
ROWS_PER_W = N // NW
R = 125
G = ROWS_PER_W // R
RD = R * D


def _take16(x, idx):
    dn = lax.GatherDimensionNumbers(
        offset_dims=(), collapsed_slice_dims=(0,), start_index_map=(0,))
    return lax.gather(x, idx[:, None], dn, slice_sizes=(1,),
                      mode=lax.GatherScatterMode.PROMISE_IN_BOUNDS)


def _lanesum(v, perms):
    for p in perms:
        v = v + _take16(v, p)
    return v


def _rsqrt(x):
    i = plsc.bitcast(x, jnp.int32)
    i = jnp.int32(0x5F3759DF) - (i >> 1)
    y = plsc.bitcast(i, jnp.float32)
    xh = x * 0.5
    for _ in range(3):
        y = y * (1.5 - xh * y * y)
    return y


def _body(feat_hbm, scale_hbm, offset_hbm, out_hbm,
          in_v, out_v, so_v, si0, si1, so0, so1):
    c = lax.axis_index("c")
    s = lax.axis_index("s")
    wid = s * NC + c
    base = wid * ROWS_PER_W

    pltpu.sync_copy(scale_hbm, so_v.at[pl.ds(0, D)])
    pltpu.sync_copy(offset_hbm, so_v.at[pl.ds(D, D)])
    sc = [so_v[pl.ds(j * L, L)] for j in range(NJ)]
    of = [so_v[pl.ds(D + j * L, L)] for j in range(NJ)]

    iota = lax.iota(jnp.int32, L)
    perms = [iota ^ 8, iota ^ 4, iota ^ 2, iota ^ 1]
    inv_d = jnp.float32(1.0 / D)
    sin = (si0, si1)
    sout = (so0, so1)

    def in_copy(g, slot):
        return pltpu.make_async_copy(
            feat_hbm.at[pl.ds((base + g * R) * D, RD)],
            in_v.at[pl.ds(slot * RD, RD)], sin[slot])

    def out_copy(g, slot):
        return pltpu.make_async_copy(
            out_v.at[pl.ds(slot * RD, RD)],
            out_hbm.at[pl.ds((base + g * R) * D, RD)], sout[slot])

    def compute(slot):
        ib = slot * RD

        @plsc.parallel_loop(0, R, unroll=4)
        def row(r):
            b = ib + r * D
            v = [in_v[pl.ds(b + j * L, L)] for j in range(NJ)]
            tot = (v[0] + v[1]) + (v[2] + v[3]) + ((v[4] + v[5]) + (v[6] + v[7]))
            mean = _lanesum(tot, perms) * inv_d
            d = [vj - mean for vj in v]
            sq = (d[0] * d[0] + d[1] * d[1]) + (d[2] * d[2] + d[3] * d[3]) + (
                (d[4] * d[4] + d[5] * d[5]) + (d[6] * d[6] + d[7] * d[7]))
            var = _lanesum(sq, perms) * inv_d + 1e-9
            rs = _rsqrt(var)
            for j in range(NJ):
                out_v[pl.ds(b + j * L, L)] = d[j] * (rs * sc[j]) + of[j]

    in_copy(0, 0).start()

    def pair(i, carry):
        for b in (0, 1):
            g = 2 * i + b
            in_copy(g + 1, 1 - b).start()
            in_copy(g, b).wait()

            @pl.when(g >= 2)
            def _():
                out_copy(g - 2, b).wait()

            compute(b)
            out_copy(g, b).start()
        return carry

    lax.fori_loop(0, (G - 1) // 2, pair, 0)

    g = G - 1
    in_copy(g, 0).wait()
    out_copy(g - 2, 0).wait()
    compute(0)
    out_copy(g, 0).start()
    out_copy(g - 1, 1).wait()
    out_copy(g, 0).wait()


@jax.jit
def _norm(feat_flat, scale_flat, offset_flat):
    mesh = plsc.VectorSubcoreMesh(core_axis_name="c", subcore_axis_name="s")
    f = pl.kernel(
        _body,
        out_type=jax.ShapeDtypeStruct((N * D,), jnp.float32),
        mesh=mesh,
        scratch_types=[
            pltpu.VMEM((2 * RD,), jnp.float32),
            pltpu.VMEM((2 * RD,), jnp.float32),
            pltpu.VMEM((2 * D,), jnp.float32),
            pltpu.SemaphoreType.DMA,
            pltpu.SemaphoreType.DMA,
            pltpu.SemaphoreType.DMA,
            pltpu.SemaphoreType.DMA,
        ],
        compiler_params=pltpu.CompilerParams(needs_layout_passes=False),
    )
    return f(feat_flat, scale_flat, offset_flat)


def kernel(feat, sizes_subg, scale, offset):
    out = _norm(feat.reshape(-1), scale.reshape(-1), offset.reshape(-1))
    return out.reshape(feat.shape)
